# baseline (speedup 1.0000x reference)
import functools

import jax
import jax.numpy as jnp
from jax import lax
from jax.experimental import pallas as pl
from jax.experimental.pallas import tpu as pltpu
from jax.experimental.pallas import tpu_sc as plsc

B = 16384
D = 32
L = 16
NC = 2
NS = 16
NW = NC * NS
BPW = B // NW
CHUNK = 128
NCHUNK = BPW // CHUNK


def _fire_feature(ue_ref, ie_ref, uidx, iidx, uval, ival, sem, f):
    copies = []
    for j in range(NCHUNK):
        sl = pl.ds(j * CHUNK, CHUNK)
        copies.append(
            pltpu.async_copy(ue_ref.at[f].at[uidx.at[j]], uval.at[sl], sem))
        copies.append(
            pltpu.async_copy(ie_ref.at[f].at[iidx.at[j]], ival.at[sl], sem))
    return copies


def _body(user_ref, item_ref, ue_ref, ie_ref, ub_ref, ib_ref, out_ref,
          uidx, iidx, uval0, ival0, uval1, ival1, acc, sem0, sem1):
    wid = lax.axis_index("s") * NC + lax.axis_index("c")

    pltpu.sync_copy(user_ref.at[wid], uidx)
    pltpu.sync_copy(item_ref.at[wid], iidx)

    bias_copies = []
    for j in range(NCHUNK):
        sl = pl.ds(j * CHUNK, CHUNK)
        bias_copies.append(
            pltpu.async_copy(ub_ref.at[uidx.at[j]], uval0.at[sl], sem0))
        bias_copies.append(
            pltpu.async_copy(ib_ref.at[iidx.at[j]], ival0.at[sl], sem0))
    for c in bias_copies:
        c.wait()
    for g in range(BPW // L):
        s = pl.ds(g * L, L)
        acc[s] = uval0[s] + ival0[s]

    bufs = ((uval0, ival0, sem0), (uval1, ival1, sem1))
    pend = _fire_feature(ue_ref, ie_ref, uidx, iidx, uval0, ival0, sem0, 0)
    for f in range(D):
        uv, iv, _ = bufs[f % 2]
        if f + 1 < D:
            nuv, niv, nsem = bufs[(f + 1) % 2]
            nxt = _fire_feature(ue_ref, ie_ref, uidx, iidx, nuv, niv, nsem,
                                f + 1)
        else:
            nxt = []
        for c in pend:
            c.wait()
        for g in range(BPW // L):
            s = pl.ds(g * L, L)
            acc[s] = acc[s] + uv[s] * iv[s]
        pend = nxt

    pltpu.sync_copy(acc, out_ref.at[pl.ds(wid * BPW, BPW)])


def kernel(user, item, user_emb, item_emb, user_bias, item_bias):
    user_r = user.astype(jnp.int32).reshape(NW, NCHUNK, CHUNK)
    item_r = item.astype(jnp.int32).reshape(NW, NCHUNK, CHUNK)
    ue_t = user_emb.T
    ie_t = item_emb.T
    ub = user_bias.reshape(-1)
    ib = item_bias.reshape(-1)
    mesh = plsc.VectorSubcoreMesh(core_axis_name="c", subcore_axis_name="s")
    k = functools.partial(
        pl.kernel,
        mesh=mesh,
        compiler_params=pltpu.CompilerParams(
            needs_layout_passes=False, use_tc_tiling_on_sc=False),
        out_type=jax.ShapeDtypeStruct((B,), jnp.float32),
        scratch_types=[
            pltpu.VMEM((NCHUNK, CHUNK), jnp.int32),
            pltpu.VMEM((NCHUNK, CHUNK), jnp.int32),
            pltpu.VMEM((BPW,), jnp.float32),
            pltpu.VMEM((BPW,), jnp.float32),
            pltpu.VMEM((BPW,), jnp.float32),
            pltpu.VMEM((BPW,), jnp.float32),
            pltpu.VMEM((BPW,), jnp.float32),
            pltpu.SemaphoreType.DMA,
            pltpu.SemaphoreType.DMA,
        ],
    )(_body)
    return k(user_r, item_r, ue_t, ie_t, ub, ib)
